# baseline (speedup 1.0000x reference)
import math

import jax
import jax.numpy as jnp
from jax import lax
from jax.experimental import pallas as pl
from jax.experimental.pallas import tpu as pltpu
from jax.experimental.pallas import tpu_sc as plsc

SOFTMAX_DIM = 32000
PADDING_TOKEN = 0
SMOOTHING_FACTOR = 0.1
CONFIDENCE = 1.0 - SMOOTHING_FACTOR
REDIST = SMOOTHING_FACTOR / (SOFTMAX_DIM - 2)
N_TOKENS = 2048
K_CONST = (SOFTMAX_DIM - 2) * REDIST * math.log(REDIST) + CONFIDENCE * math.log(CONFIDENCE)

LANES = 128
ROWS_FLAT = N_TOKENS * (SOFTMAX_DIM // LANES)
ROW_STRIDE = SOFTMAX_DIM // LANES

NW = 32
TOK_PER_W = N_TOKENS // NW

SC_ROWS = 1024
SC_ROWS_PER_W = SC_ROWS // NW
TC_ROWS = N_TOKENS - SC_ROWS

R_BLK = 128
G_TC = TC_ROWS // R_BLK
G_OFF = SC_ROWS // R_BLK

VEC_PER_ROW = SOFTMAX_DIM // 16
UNROLL = 16


def _sc_kernel(x_hbm, x1d_hbm, tgt_hbm, out_hbm,
               tgt_v, idx_v, rows_v, acc_v, mask_v, row_buf0, row_buf1,
               sem_g, sem0, sem1):
    wid = lax.axis_index("s") * 2 + lax.axis_index("c")

    zeros = jnp.zeros((16,), jnp.float32)
    acc_v[...] = zeros

    base = wid * TOK_PER_W
    pltpu.sync_copy(tgt_hbm.at[pl.ds(base, TOK_PER_W)], tgt_v)
    for j in range(TOK_PER_W // 16):
        t = tgt_v[pl.ds(j * 16, 16)]
        row_ids = (base + j * 16) + lax.iota(jnp.int32, 16)
        idx_v[pl.ds(j * 16, 16)] = row_ids * ROW_STRIDE + lax.shift_right_logical(t, 7)
        idx_v[pl.ds(TOK_PER_W + j * 16, 16)] = row_ids * ROW_STRIDE
    pltpu.async_copy(x_hbm.at[idx_v], rows_v, sem_g).wait()
    for j in range(TOK_PER_W // 16):
        t = tgt_v[pl.ds(j * 16, 16)]
        lane = lax.bitwise_and(t, 127)
        row_local = (j * 16) + lax.iota(jnp.int32, 16)
        gv = plsc.load_gather(rows_v, [row_local, lane])
        x0 = plsc.load_gather(
            rows_v, [TOK_PER_W + row_local, jnp.zeros((16,), jnp.int32)]
        )
        m = jnp.where(t != PADDING_TOKEN, 1.0, 0.0).astype(jnp.float32)
        acc_v[...] = acc_v[...] + m * (
            K_CONST + REDIST * x0 - (CONFIDENCE - REDIST) * gv
        )

    rbase = wid * SC_ROWS_PER_W
    pltpu.sync_copy(
        tgt_hbm.at[pl.ds(rbase, SC_ROWS_PER_W)], tgt_v.at[pl.ds(0, SC_ROWS_PER_W)]
    )
    for j in range(SC_ROWS_PER_W // 16):
        t = tgt_v[pl.ds(j * 16, 16)]
        mask_v[pl.ds(j * 16, 16)] = jnp.where(t != PADDING_TOKEN, -REDIST, 0.0)

    bufs = (row_buf0, row_buf1)
    sems = (sem0, sem1)

    def _issue(c, b):
        pltpu.async_copy(
            x1d_hbm.at[pl.ds((rbase + c) * SOFTMAX_DIM, SOFTMAX_DIM)], bufs[b], sems[b]
        )

    def _reduce_row(c, b):
        buf = bufs[b]
        pltpu.make_async_copy(
            x1d_hbm.at[pl.ds(0, SOFTMAX_DIM)], buf, sems[b]
        ).wait()

        def inner(j, a):
            vals = [buf[pl.ds((j * UNROLL + u) * 16, 16)] for u in range(UNROLL)]
            while len(vals) > 1:
                vals = [vals[k] + vals[k + 1] for k in range(0, len(vals), 2)]
            return a + vals[0]

        rvec = lax.fori_loop(0, VEC_PER_ROW // UNROLL, inner, zeros)
        mv = plsc.load_gather(mask_v, [jnp.broadcast_to(c, (16,)).astype(jnp.int32)])
        acc_v[...] = acc_v[...] + mv * rvec

    _issue(0, 0)
    _issue(1, 1)

    def body(k, carry):
        c = k * 2
        _reduce_row(c, 0)

        @pl.when(c + 2 < SC_ROWS_PER_W)
        def _():
            _issue(c + 2, 0)

        _reduce_row(c + 1, 1)

        @pl.when(c + 3 < SC_ROWS_PER_W)
        def _():
            _issue(c + 3, 1)

        return carry

    lax.fori_loop(0, SC_ROWS_PER_W // 2, body, 0)

    pltpu.sync_copy(acc_v, out_hbm.at[pl.ds(wid * 16, 16)])


def _sc_part(x_flat, x_1d, tgt):
    mesh = plsc.VectorSubcoreMesh(core_axis_name="c", subcore_axis_name="s")
    return pl.kernel(
        _sc_kernel,
        mesh=mesh,
        compiler_params=pltpu.CompilerParams(needs_layout_passes=False),
        out_type=jax.ShapeDtypeStruct((NW * 16,), jnp.float32),
        scratch_types=[
            pltpu.VMEM((TOK_PER_W,), jnp.int32),
            pltpu.VMEM((2 * TOK_PER_W,), jnp.int32),
            pltpu.VMEM((2 * TOK_PER_W, LANES), jnp.float32),
            pltpu.VMEM((16,), jnp.float32),
            pltpu.VMEM((SC_ROWS_PER_W,), jnp.float32),
            pltpu.VMEM((SOFTMAX_DIM,), jnp.float32),
            pltpu.VMEM((SOFTMAX_DIM,), jnp.float32),
            pltpu.SemaphoreType.DMA,
            pltpu.SemaphoreType.DMA,
            pltpu.SemaphoreType.DMA,
        ],
    )(x_flat, x_1d, tgt)


def _tc_body(x_ref, tgt_ref, out_ref):
    i = pl.program_id(0)
    rs = jnp.sum(x_ref[...], axis=1)
    t = tgt_ref[0, 0, :]
    partial = jnp.sum(jnp.where(t != PADDING_TOKEN, -REDIST * rs, 0.0))

    @pl.when(i == 0)
    def _init():
        out_ref[0, 0] = 0.0

    out_ref[0, 0] += partial


def _tc_part(x, tgt3):
    return pl.pallas_call(
        _tc_body,
        grid=(G_TC,),
        in_specs=[
            pl.BlockSpec((R_BLK, SOFTMAX_DIM), lambda i: (i + G_OFF, 0)),
            pl.BlockSpec((1, 1, R_BLK), lambda i: (i + G_OFF, 0, 0)),
        ],
        out_specs=pl.BlockSpec(memory_space=pltpu.SMEM),
        out_shape=jax.ShapeDtypeStruct((1, 1), jnp.float32),
    )(x, tgt3)


def kernel(x, tgt_tokens):
    tgt = tgt_tokens.astype(jnp.int32)
    sc_part = _sc_part(x.reshape(ROWS_FLAT, LANES), x.reshape(-1), tgt)
    tgt3 = tgt.reshape(N_TOKENS // R_BLK, 1, R_BLK)
    tc_part = _tc_part(x, tgt3)
    return tc_part[0, 0] + jnp.sum(sc_part)


# speedup vs baseline: 4.5456x; 4.5456x over previous
import math

import jax
import jax.numpy as jnp
from jax import lax
from jax.experimental import pallas as pl
from jax.experimental.pallas import tpu as pltpu
from jax.experimental.pallas import tpu_sc as plsc

SOFTMAX_DIM = 32000
PADDING_TOKEN = 0
SMOOTHING_FACTOR = 0.1
CONFIDENCE = 1.0 - SMOOTHING_FACTOR
REDIST = SMOOTHING_FACTOR / (SOFTMAX_DIM - 2)
N_TOKENS = 2048
K_CONST = (SOFTMAX_DIM - 2) * REDIST * math.log(REDIST) + CONFIDENCE * math.log(CONFIDENCE)

LANES = 128
NW = 32
TOK_PER_W = N_TOKENS // NW

R_BLK = 128
G_BLK = N_TOKENS // R_BLK


def _sc_gather_kernel(x_hbm, tgt_hbm, out_hbm, tgt_v, rows_v, acc_v, sem):
    wid = lax.axis_index("s") * 2 + lax.axis_index("c")
    base = wid * TOK_PER_W
    pltpu.sync_copy(tgt_hbm.at[pl.ds(base, TOK_PER_W)], tgt_v)

    for jo in range(TOK_PER_W // 16):
        tv = tgt_v[pl.ds(jo * 16, 16)]
        cbv = tv - lax.bitwise_and(tv, 127)
        for ji in range(16):
            j = jo * 16 + ji
            tile_row = base + (j // 8) * 8
            cb = pl.multiple_of(cbv[ji], LANES)
            pltpu.async_copy(
                x_hbm.at[pl.ds(tile_row, 8), pl.ds(cb, LANES)],
                rows_v.at[j],
                sem,
            )
    for j in range(TOK_PER_W):
        pltpu.make_async_copy(
            x_hbm.at[pl.ds(0, 8), pl.ds(0, LANES)], rows_v.at[j], sem
        ).wait()

    acc = jnp.zeros((16,), jnp.float32)
    for j in range(TOK_PER_W // 16):
        t = tgt_v[pl.ds(j * 16, 16)]
        lane = lax.bitwise_and(t, 127)
        row_local = (j * 16) + lax.iota(jnp.int32, 16)
        sub_row = lax.bitwise_and(lax.iota(jnp.int32, 16), 7)
        gv = plsc.load_gather(rows_v, [row_local, sub_row, lane])
        m = jnp.where(t != PADDING_TOKEN, 1.0, 0.0).astype(jnp.float32)
        acc = acc + m * gv
    acc_v[...] = acc
    pltpu.sync_copy(acc_v, out_hbm.at[pl.ds(wid * 16, 16)])


def _sc_gather(x, tgt):
    mesh = plsc.VectorSubcoreMesh(core_axis_name="c", subcore_axis_name="s")
    return pl.kernel(
        _sc_gather_kernel,
        mesh=mesh,
        compiler_params=pltpu.CompilerParams(
            needs_layout_passes=False, use_tc_tiling_on_sc=True
        ),
        out_type=jax.ShapeDtypeStruct((NW * 16,), jnp.float32),
        scratch_types=[
            pltpu.VMEM((TOK_PER_W,), jnp.int32),
            pltpu.VMEM((TOK_PER_W, 8, LANES), jnp.float32),
            pltpu.VMEM((16,), jnp.float32),
            pltpu.SemaphoreType.DMA,
        ],
    )(x, tgt)


def _tc_body(x_ref, tgt_ref, out_ref):
    i = pl.program_id(0)
    x = x_ref[...]
    rs = jnp.sum(x, axis=1)
    x0 = x[:, 0]
    t = tgt_ref[0, 0, :]
    partial = jnp.sum(
        jnp.where(t != PADDING_TOKEN, K_CONST + REDIST * (x0 - rs), 0.0)
    )

    @pl.when(i == 0)
    def _init():
        out_ref[0, 0] = 0.0

    out_ref[0, 0] += partial


def _tc_part(x, tgt3):
    return pl.pallas_call(
        _tc_body,
        grid=(G_BLK,),
        in_specs=[
            pl.BlockSpec((R_BLK, SOFTMAX_DIM), lambda i: (i, 0)),
            pl.BlockSpec((1, 1, R_BLK), lambda i: (i, 0, 0)),
        ],
        out_specs=pl.BlockSpec(memory_space=pltpu.SMEM),
        out_shape=jax.ShapeDtypeStruct((1, 1), jnp.float32),
    )(x, tgt3)


def kernel(x, tgt_tokens):
    tgt = tgt_tokens.astype(jnp.int32)
    sg = _sc_gather(x, tgt)
    tgt3 = tgt.reshape(G_BLK, 1, R_BLK)
    tc_part = _tc_part(x, tgt3)
    return tc_part[0, 0] - (CONFIDENCE - REDIST) * jnp.sum(sg)


# speedup vs baseline: 4.5684x; 1.0050x over previous
import math

import jax
import jax.numpy as jnp
from jax import lax
from jax.experimental import pallas as pl
from jax.experimental.pallas import tpu as pltpu
from jax.experimental.pallas import tpu_sc as plsc

SOFTMAX_DIM = 32000
PADDING_TOKEN = 0
SMOOTHING_FACTOR = 0.1
CONFIDENCE = 1.0 - SMOOTHING_FACTOR
REDIST = SMOOTHING_FACTOR / (SOFTMAX_DIM - 2)
N_TOKENS = 2048
K_CONST = (SOFTMAX_DIM - 2) * REDIST * math.log(REDIST) + CONFIDENCE * math.log(CONFIDENCE)

LANES = 128
NW = 32
TOK_PER_W = N_TOKENS // NW

R_BLK = 64
G_BLK = N_TOKENS // R_BLK


def _sc_gather_kernel(x_hbm, tgt_hbm, out_hbm, tgt_v, rows_v, acc_v, sem):
    wid = lax.axis_index("s") * 2 + lax.axis_index("c")
    base = wid * TOK_PER_W
    pltpu.sync_copy(tgt_hbm.at[pl.ds(base, TOK_PER_W)], tgt_v)

    for jo in range(TOK_PER_W // 16):
        tv = tgt_v[pl.ds(jo * 16, 16)]
        cbv = tv - lax.bitwise_and(tv, 127)
        for ji in range(16):
            j = jo * 16 + ji
            tile_row = base + (j // 8) * 8
            cb = pl.multiple_of(cbv[ji], LANES)
            pltpu.async_copy(
                x_hbm.at[pl.ds(tile_row, 8), pl.ds(cb, LANES)],
                rows_v.at[j],
                sem,
            )
    for j in range(TOK_PER_W):
        pltpu.make_async_copy(
            x_hbm.at[pl.ds(0, 8), pl.ds(0, LANES)], rows_v.at[j], sem
        ).wait()

    acc = jnp.zeros((16,), jnp.float32)
    for j in range(TOK_PER_W // 16):
        t = tgt_v[pl.ds(j * 16, 16)]
        lane = lax.bitwise_and(t, 127)
        row_local = (j * 16) + lax.iota(jnp.int32, 16)
        sub_row = lax.bitwise_and(lax.iota(jnp.int32, 16), 7)
        gv = plsc.load_gather(rows_v, [row_local, sub_row, lane])
        m = jnp.where(t != PADDING_TOKEN, 1.0, 0.0).astype(jnp.float32)
        acc = acc + m * gv
    acc_v[...] = acc
    pltpu.sync_copy(acc_v, out_hbm.at[pl.ds(wid * 16, 16)])


def _sc_gather(x, tgt):
    mesh = plsc.VectorSubcoreMesh(core_axis_name="c", subcore_axis_name="s")
    return pl.kernel(
        _sc_gather_kernel,
        mesh=mesh,
        compiler_params=pltpu.CompilerParams(
            needs_layout_passes=False, use_tc_tiling_on_sc=True
        ),
        out_type=jax.ShapeDtypeStruct((NW * 16,), jnp.float32),
        scratch_types=[
            pltpu.VMEM((TOK_PER_W,), jnp.int32),
            pltpu.VMEM((TOK_PER_W, 8, LANES), jnp.float32),
            pltpu.VMEM((16,), jnp.float32),
            pltpu.SemaphoreType.DMA,
        ],
    )(x, tgt)


def _tc_body(x_ref, tgt_ref, out_ref):
    i = pl.program_id(0)
    x = x_ref[...]
    rs = jnp.sum(x, axis=1)
    x0 = x[:, 0]
    t = tgt_ref[0, 0, :]
    partial = jnp.sum(
        jnp.where(t != PADDING_TOKEN, K_CONST + REDIST * (x0 - rs), 0.0)
    )

    @pl.when(i == 0)
    def _init():
        out_ref[0, 0] = 0.0

    out_ref[0, 0] += partial


def _tc_part(x, tgt3):
    return pl.pallas_call(
        _tc_body,
        grid=(G_BLK,),
        in_specs=[
            pl.BlockSpec((R_BLK, SOFTMAX_DIM), lambda i: (i, 0)),
            pl.BlockSpec((1, 1, R_BLK), lambda i: (i, 0, 0)),
        ],
        out_specs=pl.BlockSpec(memory_space=pltpu.SMEM),
        out_shape=jax.ShapeDtypeStruct((1, 1), jnp.float32),
    )(x, tgt3)


def kernel(x, tgt_tokens):
    tgt = tgt_tokens.astype(jnp.int32)
    sg = _sc_gather(x, tgt)
    tgt3 = tgt.reshape(G_BLK, 1, R_BLK)
    tc_part = _tc_part(x, tgt3)
    return tc_part[0, 0] - (CONFIDENCE - REDIST) * jnp.sum(sg)
